# Initial kernel scaffold; baseline (speedup 1.0000x reference)
import jax
import jax.numpy as jnp
from jax.experimental import pallas as pl


def kernel(x, edge_index, gin_w, gin_b, ggc_weight, ggc_wih, ggc_whh, ggc_bih, ggc_bhh, gn_weight, gn_bias, gn_mean_scale, lin_w, lin_b):
    raise NotImplementedError("write your pallas kernel here")


# speedup vs baseline: 9.4083x; 9.4083x over previous
import functools

import jax
import jax.numpy as jnp
from jax import lax
from jax.experimental import pallas as pl
from jax.experimental.pallas import tpu as pltpu
from jax.experimental.pallas import tpu_sc as plsc

NC = 2
NS = 16
LANES = 16
CH = 128
KB = 16
KBG = 8
ZROWS = 1024


def _npad(n):
    return ((n + 1 + NS * 8 - 1) // (NS * 8)) * (NS * 8)


def _mesh():
    return plsc.VectorSubcoreMesh(
        core_axis_name="c", subcore_axis_name="s", num_cores=NC, num_subcores=NS
    )


@functools.lru_cache(maxsize=None)
def _sc_scatter(n, rows_tot):
    rpt = rows_tot // NS
    nblk = rpt // KB
    npad = _npad(n)
    half = n // 2
    hpad = ((half + NS * 8 - 1) // (NS * 8)) * (NS * 8)
    accr = hpad + 64
    zr = accr // NS
    nzf, zt = zr // ZROWS, zr % ZROWS
    orows = half // NS

    @functools.partial(
        pl.kernel,
        out_type=jax.ShapeDtypeStruct((NC * npad, LANES), jnp.float32),
        mesh=_mesh(),
        compiler_params=pltpu.CompilerParams(use_tc_tiling_on_sc=False),
        scratch_types=[
            pltpu.VMEM((KB, CH), jnp.int32),
            pltpu.VMEM((KB, CH), jnp.int32),
            pltpu.VMEM((KB, CH, LANES), jnp.float32),
            pltpu.VMEM((ZROWS, LANES), jnp.float32),
            pltpu.VMEM_SHARED((accr, LANES), jnp.float32),
            pltpu.SemaphoreType.DMA,
            pltpu.SemaphoreType.DMA,
        ],
    )
    def sc_step(m_hbm, src_hbm, dst_hbm, out_hbm, srcb, dstb, rows, zbuf, acc,
                gsem, ssem):
        c = lax.axis_index("c")
        s = lax.axis_index("s")
        zero = jnp.zeros((LANES,), jnp.float32)

        def zb(i, _):
            zbuf[i] = zero
            return ()

        lax.fori_loop(0, ZROWS, zb, ())
        base_row = s * rpt

        for p in range(2):
            nbase = p * half
            zbase = s * zr

            def zcp(i, _):
                pltpu.sync_copy(zbuf, acc.at[pl.ds(zbase + i * ZROWS, ZROWS)])
                return ()

            lax.fori_loop(0, nzf, zcp, ())
            if zt:
                pltpu.sync_copy(zbuf.at[pl.ds(0, zt)],
                                acc.at[pl.ds(zbase + nzf * ZROWS, zt)])
            plsc.subcore_barrier()

            def blk(i, _):
                row0 = base_row + i * KB
                pltpu.sync_copy(src_hbm.at[c, pl.ds(row0, KB)], srcb)
                pltpu.sync_copy(dst_hbm.at[pl.ds(row0, KB)], dstb)

                def remap(j, _):
                    drow = dstb.at[j]
                    for k in range(CH // LANES):
                        d = drow[pl.ds(k * LANES, LANES)]
                        loc = d - nbase
                        ok = (loc >= 0) & (loc < half)
                        dump = hpad + (d & 63)
                        drow[pl.ds(k * LANES, LANES)] = jnp.where(ok, loc, dump)
                    return ()

                lax.fori_loop(0, KB, remap, ())
                g = [pltpu.async_copy(m_hbm.at[srcb.at[j]], rows.at[j], gsem)
                     for j in range(KB)]
                for d in g:
                    d.wait()
                sc = [pltpu.async_copy(rows.at[j], acc.at[dstb.at[j]], ssem,
                                       add=True) for j in range(KB)]
                for d in sc:
                    d.wait()
                return ()

            lax.fori_loop(0, nblk, blk, ())
            plsc.subcore_barrier()
            pltpu.sync_copy(
                acc.at[pl.ds(s * orows, orows)],
                out_hbm.at[pl.ds(c * npad + nbase + s * orows, orows)])
            plsc.subcore_barrier()

    return sc_step


@functools.lru_cache(maxsize=None)
def _sc_gin(n, rows_tot):
    rps = rows_tot // NC
    rpt = rps // NS
    nblk = rpt // KBG
    npad = _npad(n)
    zr = npad // NS

    @functools.partial(
        pl.kernel,
        out_type=jax.ShapeDtypeStruct((NC * npad,), jnp.float32),
        mesh=_mesh(),
        compiler_params=pltpu.CompilerParams(use_tc_tiling_on_sc=False),
        scratch_types=[
            pltpu.VMEM((KBG, CH), jnp.int32),
            pltpu.VMEM((KBG, CH), jnp.int32),
            pltpu.VMEM((KBG, CH), jnp.float32),
            pltpu.VMEM((zr,), jnp.float32),
            pltpu.VMEM_SHARED((npad,), jnp.float32),
            pltpu.SemaphoreType.DMA,
            pltpu.SemaphoreType.DMA,
        ],
    )
    def gin(x_hbm, src_hbm, dst_hbm, out_hbm, srcb, dstb, valb, zbuf,
            acc, gsem, ssem):
        c = lax.axis_index("c")
        s = lax.axis_index("s")
        zero = jnp.zeros((LANES,), jnp.float32)

        def zb(i, _):
            zbuf[pl.ds(i * LANES, LANES)] = zero
            return ()

        lax.fori_loop(0, zr // LANES, zb, ())
        pltpu.sync_copy(zbuf, acc.at[pl.ds(s * zr, zr)])
        plsc.subcore_barrier()

        base_row = c * rps + s * rpt

        def blk(i, _):
            row0 = base_row + i * KBG
            pltpu.sync_copy(src_hbm.at[pl.ds(row0, KBG)], srcb)
            pltpu.sync_copy(dst_hbm.at[pl.ds(row0, KBG)], dstb)
            g = [pltpu.async_copy(x_hbm.at[srcb.at[j]], valb.at[j], gsem)
                 for j in range(KBG)]
            for d in g:
                d.wait()
            sc = [pltpu.async_copy(valb.at[j], acc.at[dstb.at[j]], ssem,
                                   add=True) for j in range(KBG)]
            for d in sc:
                d.wait()
            return ()

        lax.fori_loop(0, nblk, blk, ())
        plsc.subcore_barrier()
        pltpu.sync_copy(acc.at[pl.ds(s * zr, zr)], zbuf)
        pltpu.sync_copy(zbuf, out_hbm.at[pl.ds(c * npad + s * zr, zr)])

    return gin


@functools.lru_cache(maxsize=None)
def _tc_gin_post(n, f, bn, interpret=False):
    nb = n // bn

    def body(x_ref, a_ref, w_ref, b_ref, x1_ref, ps_ref, pq_ref):
        a = a_ref[...]
        sv = x_ref[...][:, 0] + a[:, 0] + a[:, 1]
        x1 = jnp.maximum(sv[:, None] * w_ref[...][0][None, :] + b_ref[...], 0.0)
        x1_ref[...] = x1

        @pl.when(pl.program_id(0) == 0)
        def _():
            ps_ref[...] = jnp.zeros((8, f), jnp.float32)
            pq_ref[...] = jnp.zeros((8, f), jnp.float32)

        ps_ref[...] += jnp.sum(x1.reshape(bn // 8, 8, f), axis=0)
        pq_ref[...] += jnp.sum((x1 * x1).reshape(bn // 8, 8, f), axis=0)

    return pl.pallas_call(
        body,
        grid=(nb,),
        in_specs=[
            pl.BlockSpec((bn, 1), lambda i: (i, 0)),
            pl.BlockSpec((bn, NC), lambda i: (i, 0)),
            pl.BlockSpec((1, f), lambda i: (0, 0)),
            pl.BlockSpec((1, f), lambda i: (0, 0)),
        ],
        out_specs=[
            pl.BlockSpec((bn, f), lambda i: (i, 0)),
            pl.BlockSpec((8, f), lambda i: (0, 0)),
            pl.BlockSpec((8, f), lambda i: (0, 0)),
        ],
        out_shape=[
            jax.ShapeDtypeStruct((n, f), jnp.float32),
            jax.ShapeDtypeStruct((8, f), jnp.float32),
            jax.ShapeDtypeStruct((8, f), jnp.float32),
        ],
        interpret=interpret,
    )


@functools.lru_cache(maxsize=None)
def _tc_norm_mm(n, f, bn, interpret=False):
    nb = n // bn

    def body(x1_ref, ps_ref, pq_ref, gw_ref, gb_ref, gm_ref, w_ref,
             x2_ref, m_ref):
        mean = jnp.sum(ps_ref[...], axis=0) / n
        m2 = jnp.sum(pq_ref[...], axis=0) / n
        mm = mean * gm_ref[...][0]
        var = m2 - 2.0 * mm * mean + mm * mm
        a = gw_ref[...][0] * jax.lax.rsqrt(var + 1e-5)
        b = gb_ref[...][0] - mm * a
        x2 = x1_ref[...] * a[None, :] + b[None, :]
        x2_ref[...] = x2
        m = jnp.dot(x2, w_ref[...], preferred_element_type=jnp.float32)
        m_ref[...] = jnp.stack([m[:, :LANES], m[:, LANES:]], axis=0)

    return pl.pallas_call(
        body,
        grid=(nb,),
        in_specs=[
            pl.BlockSpec((bn, f), lambda i: (i, 0)),
            pl.BlockSpec((8, f), lambda i: (0, 0)),
            pl.BlockSpec((8, f), lambda i: (0, 0)),
            pl.BlockSpec((1, f), lambda i: (0, 0)),
            pl.BlockSpec((1, f), lambda i: (0, 0)),
            pl.BlockSpec((1, f), lambda i: (0, 0)),
            pl.BlockSpec((f, f), lambda i: (0, 0)),
        ],
        out_specs=[
            pl.BlockSpec((bn, f), lambda i: (i, 0)),
            pl.BlockSpec((NC, bn, LANES), lambda i: (0, i, 0)),
        ],
        out_shape=[
            jax.ShapeDtypeStruct((n, f), jnp.float32),
            jax.ShapeDtypeStruct((NC, n, LANES), jnp.float32),
        ],
        interpret=interpret,
    )


def _gru_math(agg, h, wih_t, whh_t, bih, bhh, f):
    m = jnp.concatenate([agg[0], agg[1]], axis=1)
    gi = jnp.dot(m, wih_t, preferred_element_type=jnp.float32) + bih
    gh = jnp.dot(h, whh_t, preferred_element_type=jnp.float32) + bhh
    r = jax.nn.sigmoid(gi[:, :f] + gh[:, :f])
    z = jax.nn.sigmoid(gi[:, f:2 * f] + gh[:, f:2 * f])
    nn = jnp.tanh(gi[:, 2 * f:] + r * gh[:, 2 * f:])
    return (1.0 - z) * nn + z * h


@functools.lru_cache(maxsize=None)
def _tc_gru_a(n, f, bn, interpret=False):
    nb = n // bn

    def body(a_ref, h_ref, wi_ref, wh_ref, bi_ref, bh_ref, wn_ref,
             h1_ref, m_ref):
        h1 = _gru_math(a_ref[...], h_ref[...], wi_ref[...], wh_ref[...],
                       bi_ref[...], bh_ref[...], f)
        h1_ref[...] = h1
        m = jnp.dot(h1, wn_ref[...], preferred_element_type=jnp.float32)
        m_ref[...] = jnp.stack([m[:, :LANES], m[:, LANES:]], axis=0)

    return pl.pallas_call(
        body,
        grid=(nb,),
        in_specs=[
            pl.BlockSpec((NC, bn, LANES), lambda i: (0, i, 0)),
            pl.BlockSpec((bn, f), lambda i: (i, 0)),
            pl.BlockSpec((f, 3 * f), lambda i: (0, 0)),
            pl.BlockSpec((f, 3 * f), lambda i: (0, 0)),
            pl.BlockSpec((1, 3 * f), lambda i: (0, 0)),
            pl.BlockSpec((1, 3 * f), lambda i: (0, 0)),
            pl.BlockSpec((f, f), lambda i: (0, 0)),
        ],
        out_specs=[
            pl.BlockSpec((bn, f), lambda i: (i, 0)),
            pl.BlockSpec((NC, bn, LANES), lambda i: (0, i, 0)),
        ],
        out_shape=[
            jax.ShapeDtypeStruct((n, f), jnp.float32),
            jax.ShapeDtypeStruct((NC, n, LANES), jnp.float32),
        ],
        interpret=interpret,
    )


@functools.lru_cache(maxsize=None)
def _tc_gru_b(n, f, bn, last, interpret=False):
    nb = n // bn

    def body(a_ref, h_ref, x_ref, wi_ref, wh_ref, bi_ref, bh_ref, wn_ref,
             bn_ref, *outs):
        h2 = _gru_math(a_ref[...], h_ref[...], wi_ref[...], wh_ref[...],
                       bi_ref[...], bh_ref[...], f)
        xn = x_ref[...] + jnp.maximum(h2, 0.0)
        if last:
            outs[0][...] = jnp.dot(
                xn, wn_ref[...], preferred_element_type=jnp.float32
            ) + bn_ref[...]
        else:
            outs[0][...] = xn
            m = jnp.dot(xn, wn_ref[...], preferred_element_type=jnp.float32)
            outs[1][...] = jnp.stack([m[:, :LANES], m[:, LANES:]], axis=0)

    out_specs = [pl.BlockSpec((bn, f), lambda i: (i, 0))]
    out_shape = [jax.ShapeDtypeStruct((n, f), jnp.float32)]
    if not last:
        out_specs.append(pl.BlockSpec((NC, bn, LANES), lambda i: (0, i, 0)))
        out_shape.append(jax.ShapeDtypeStruct((NC, n, LANES), jnp.float32))

    return pl.pallas_call(
        body,
        grid=(nb,),
        in_specs=[
            pl.BlockSpec((NC, bn, LANES), lambda i: (0, i, 0)),
            pl.BlockSpec((bn, f), lambda i: (i, 0)),
            pl.BlockSpec((bn, f), lambda i: (i, 0)),
            pl.BlockSpec((f, 3 * f), lambda i: (0, 0)),
            pl.BlockSpec((f, 3 * f), lambda i: (0, 0)),
            pl.BlockSpec((1, 3 * f), lambda i: (0, 0)),
            pl.BlockSpec((1, 3 * f), lambda i: (0, 0)),
            pl.BlockSpec((f, f), lambda i: (0, 0)),
            pl.BlockSpec((1, f), lambda i: (0, 0)),
        ],
        out_specs=out_specs,
        out_shape=out_shape,
        interpret=interpret,
    )


def kernel(x, edge_index, gin_w, gin_b, ggc_weight, ggc_wih, ggc_whh,
           ggc_bih, ggc_bhh, gn_weight, gn_bias, gn_mean_scale, lin_w, lin_b):
    n = x.shape[0]
    e = edge_index.shape[1]
    f = gin_w.shape[1]
    num_l = ggc_weight.shape[0]
    bn = 1000

    rows_tot = -(-e // CH)
    blk_unit = NS * KB
    rows_tot = -(-rows_tot // blk_unit) * blk_unit
    e_pad = rows_tot * CH
    src = edge_index[0]
    dst = edge_index[1]
    srcp = jnp.concatenate([src, jnp.zeros((e_pad - e,), jnp.int32)])
    dstp = jnp.concatenate([dst, jnp.full((e_pad - e,), n, jnp.int32)])
    srcA = srcp.reshape(rows_tot, CH)
    src2 = jnp.stack([srcA, srcA + n])
    dst2 = dstp.reshape(rows_tot, CH)

    wih_t = jnp.transpose(ggc_wih, (0, 2, 1))
    whh_t = jnp.transpose(ggc_whh, (0, 2, 1))
    bih2 = ggc_bih.reshape(num_l, 1, 3 * f)
    bhh2 = ggc_bhh.reshape(num_l, 1, 3 * f)
    gin_w2 = gin_w.reshape(1, f)
    gin_b2 = gin_b.reshape(1, f)
    gnw2 = gn_weight.reshape(1, f)
    gnb2 = gn_bias.reshape(1, f)
    gnm2 = gn_mean_scale.reshape(1, f)
    lin_b2 = lin_b.reshape(1, f)

    aggp = _sc_gin(n, rows_tot)(x.reshape(n), srcA, dst2)
    aggn = aggp.reshape(NC, _npad(n))[:, :n].T
    x1, psum, psq = _tc_gin_post(n, f, bn)(x, aggn, gin_w2, gin_b2)
    x2, m3 = _tc_norm_mm(n, f, bn)(x1, psum, psq, gnw2, gnb2, gnm2,
                                   ggc_weight[0, 0])

    h = x2
    xres = x2
    y = None
    for i in range(num_l):
        for l in range(2):
            agg = _sc_scatter(n, rows_tot)(m3.reshape(2 * n, LANES), src2, dst2)
            agg3 = agg.reshape(NC, _npad(n), LANES)
            if l == 0:
                h, m3 = _tc_gru_a(n, f, bn)(
                    agg3, h, wih_t[i], whh_t[i], bih2[i], bhh2[i],
                    ggc_weight[i, 1])
            elif i + 1 < num_l:
                xres, m3 = _tc_gru_b(n, f, bn, False)(
                    agg3, h, xres, wih_t[i], whh_t[i], bih2[i], bhh2[i],
                    ggc_weight[i + 1, 0], lin_b2)
                h = xres
            else:
                (y,) = _tc_gru_b(n, f, bn, True)(
                    agg3, h, xres, wih_t[i], whh_t[i], bih2[i], bhh2[i],
                    lin_w, lin_b2)
    return y
